# Initial kernel scaffold; baseline (speedup 1.0000x reference)
import jax
import jax.numpy as jnp
from jax.experimental import pallas as pl


def kernel(tokens, tables):
    raise NotImplementedError("write your pallas kernel here")


# speedup vs baseline: 9.5630x; 9.5630x over previous
import functools

import jax
import jax.numpy as jnp
from jax import lax
from jax.experimental import pallas as pl
from jax.experimental.pallas import tpu as pltpu
from jax.experimental.pallas import tpu_sc as plsc

_VOCAB = 1000
_EMBED_DIM = 16
_NUM_WORKERS = 32
_CHUNK = 128


@functools.lru_cache(maxsize=None)
def _make_gather(total: int):
    per_w = total // _NUM_WORKERS
    n_chunks = per_w // _CHUNK
    mesh = plsc.VectorSubcoreMesh(core_axis_name="c", subcore_axis_name="s")

    @functools.partial(
        pl.kernel,
        mesh=mesh,
        out_type=jax.ShapeDtypeStruct((total, _EMBED_DIM), jnp.float32),
        compiler_params=pltpu.CompilerParams(use_tc_tiling_on_sc=False),
        scratch_types=[
            pltpu.VMEM((n_chunks, _CHUNK), jnp.int32),
            pltpu.VMEM((_CHUNK, _EMBED_DIM), jnp.float32),
            pltpu.SemaphoreType.DMA,
        ],
    )
    def gather_kernel(table_hbm, idx_hbm, out_hbm, idx_v, rows_v, sem):
        wid = lax.axis_index("s") * 2 + lax.axis_index("c")
        pltpu.sync_copy(idx_hbm.at[wid], idx_v)
        base = wid * per_w

        def body(j, carry):
            pltpu.async_copy(table_hbm.at[idx_v.at[j]], rows_v, sem).wait()
            pltpu.sync_copy(rows_v, out_hbm.at[pl.ds(base + j * _CHUNK, _CHUNK)])
            return carry

        lax.fori_loop(0, n_chunks, body, 0)

    return gather_kernel


def kernel(tokens, tables):
    b, w, nf = tokens.shape
    total = b * w * nf
    flat_table = tables.reshape(nf * _VOCAB, _EMBED_DIM)
    offs = jnp.arange(nf, dtype=jnp.int32) * _VOCAB
    flat_idx = (tokens.astype(jnp.int32) + offs).reshape(
        _NUM_WORKERS, total // _NUM_WORKERS // _CHUNK, _CHUNK
    )
    out = _make_gather(total)(flat_table, flat_idx)
    return out.reshape(b, w, nf * _EMBED_DIM)


# speedup vs baseline: 16.1407x; 1.6878x over previous
import functools

import jax
import jax.numpy as jnp
from jax import lax
from jax.experimental import pallas as pl
from jax.experimental.pallas import tpu as pltpu
from jax.experimental.pallas import tpu_sc as plsc

_VOCAB = 1000
_EMBED_DIM = 16
_NUM_WORKERS = 32
_CHUNK = 128
_R = 13


@functools.lru_cache(maxsize=None)
def _make_gather(total: int):
    per_w = total // _NUM_WORKERS
    n_chunks = per_w // _CHUNK
    assert n_chunks % _R == 0
    mesh = plsc.VectorSubcoreMesh(core_axis_name="c", subcore_axis_name="s")

    @functools.partial(
        pl.kernel,
        mesh=mesh,
        out_type=jax.ShapeDtypeStruct((total, _EMBED_DIM), jnp.float32),
        compiler_params=pltpu.CompilerParams(use_tc_tiling_on_sc=False),
        scratch_types=[
            pltpu.VMEM((n_chunks, _CHUNK), jnp.int32),
            pltpu.VMEM((_R, _CHUNK, _EMBED_DIM), jnp.float32),
        ]
        + [pltpu.SemaphoreType.DMA] * _R,
    )
    def gather_kernel(table_hbm, idx_hbm, out_hbm, idx_v, rows_v, *sems):
        wid = lax.axis_index("s") * 2 + lax.axis_index("c")
        pltpu.sync_copy(idx_hbm.at[wid], idx_v)
        base = wid * per_w

        for r in range(_R):
            pltpu.async_copy(table_hbm.at[idx_v.at[r]], rows_v.at[r], sems[r])

        def body(g, carry):
            j0 = g * _R
            for r in range(_R):
                j = j0 + r
                pltpu.make_async_copy(
                    out_hbm.at[pl.ds(0, _CHUNK)], rows_v.at[r], sems[r]
                ).wait()
                pltpu.sync_copy(
                    rows_v.at[r], out_hbm.at[pl.ds(base + j * _CHUNK, _CHUNK)]
                )

                @pl.when(j + _R < n_chunks)
                def _():
                    pltpu.async_copy(
                        table_hbm.at[idx_v.at[j + _R]], rows_v.at[r], sems[r]
                    )

            return carry

        lax.fori_loop(0, n_chunks // _R, body, 0)

    return gather_kernel


def kernel(tokens, tables):
    b, w, nf = tokens.shape
    total = b * w * nf
    flat_table = tables.reshape(nf * _VOCAB, _EMBED_DIM)
    offs = jnp.arange(nf, dtype=jnp.int32) * _VOCAB
    flat_idx = (tokens.astype(jnp.int32) + offs).reshape(
        _NUM_WORKERS, total // _NUM_WORKERS // _CHUNK, _CHUNK
    )
    out = _make_gather(total)(flat_table, flat_idx)
    return out.reshape(b, w, nf * _EMBED_DIM)
